# baseline (speedup 1.0000x reference)
import functools

import jax
import jax.numpy as jnp
from jax import lax
from jax.experimental import pallas as pl
from jax.experimental.pallas import tpu as pltpu
from jax.experimental.pallas import tpu_sc as plsc

N = 10000
E = 320000
D = 128
OUT = 2

NC = 2
NS = 16
NW = NC * NS

CE = E // NW
B = 128
EP = 10240
K = EP // B
KP = K // 2
BC = 80
KC = CE // BC
NP = 10240
RPT = NP // NS
CPT = NP // NS


_MESH = plsc.VectorSubcoreMesh(core_axis_name="c", subcore_axis_name="s")


def _seg_body(h_hbm, src2, dst2, p_out, idx_s, idx_d, rows_a, rows_b,
              acc, sem_a, sem_b):
  cid = lax.axis_index("c")
  sid = lax.axis_index("s")
  wid = cid * NS + sid

  z16 = jnp.zeros((16,), jnp.float32)

  def zstore(i, _):
    rows_a[i // 8, pl.ds((i % 8) * 16, 16)] = z16
    return _

  lax.fori_loop(0, B * 8, zstore, None)
  for q in range(RPT // B):
    pltpu.sync_copy(rows_a, acc.at[pl.ds(sid * RPT + q * B, B), :])
  plsc.subcore_barrier()

  def gather(c, buf, sem):
    return pltpu.async_copy(h_hbm.at[idx_s.at[c]], buf, sem)

  def gwait(c, buf, sem):
    pltpu.make_async_copy(h_hbm.at[idx_s.at[c]], buf, sem).wait()

  def scatter(c, buf):
    pltpu.sync_copy(buf, acc.at[idx_d.at[c]], add=True)

  def pair(t, _):
    i = 2 * t
    gather(i + 1, rows_b, sem_b)
    gwait(i, rows_a, sem_a)
    scatter(i, rows_a)
    gather(i + 2, rows_a, sem_a)
    gwait(i + 1, rows_b, sem_b)
    scatter(i + 1, rows_b)
    return _

  def chunk(c, _):
    gather(c, rows_a, sem_a).wait()
    scatter(c, rows_a)
    return _

  for p in range(K // KP):
    pltpu.sync_copy(src2.at[wid, pl.ds(p * KP, KP), :], idx_s)
    pltpu.sync_copy(dst2.at[wid, pl.ds(p * KP, KP), :], idx_d)
    lax.fori_loop(0, KP, chunk, None)
  del pair

  plsc.subcore_barrier()
  pltpu.sync_copy(acc.at[pl.ds(sid * RPT, RPT), :],
                  p_out.at[cid, pl.ds(sid * RPT, RPT), :])


_sc_seg = pl.kernel(
    _seg_body,
    out_type=(jax.ShapeDtypeStruct((NC, NP, D), jnp.float32),),
    mesh=_MESH,
    scratch_types=[
        pltpu.VMEM((KP, B), jnp.int32),
        pltpu.VMEM((KP, B), jnp.int32),
        pltpu.VMEM((B, D), jnp.float32),
        pltpu.VMEM((B, D), jnp.float32),
        pltpu.VMEM_SHARED((NP, D), jnp.float32),
        pltpu.SemaphoreType.DMA,
        pltpu.SemaphoreType.DMA,
    ])


def _cnt_body(dst2, zrows, ones_hbm, c_out, idx_d, ones_v, cacc):
  cid = lax.axis_index("c")
  sid = lax.axis_index("s")
  wid = cid * NS + sid

  pltpu.sync_copy(dst2.at[wid], idx_d)
  pltpu.sync_copy(zrows, cacc.at[pl.ds(sid * RPT, RPT), :])
  pltpu.sync_copy(ones_hbm, ones_v)
  plsc.subcore_barrier()

  def chunk(c, _):
    pltpu.sync_copy(ones_v, cacc.at[idx_d.at[c]], add=True)
    return _

  lax.fori_loop(0, KC, chunk, None)
  plsc.subcore_barrier()

  pltpu.sync_copy(cacc.at[pl.ds(sid * RPT, RPT), :],
                  c_out.at[cid, pl.ds(sid * RPT, RPT), :])


_sc_counts = pl.kernel(
    _cnt_body,
    out_type=(jax.ShapeDtypeStruct((NC, NP, D), jnp.float32),),
    mesh=_MESH,
    scratch_types=[
        pltpu.VMEM((KC, BC), jnp.int32),
        pltpu.VMEM((BC, D), jnp.float32),
        pltpu.VMEM_SHARED((NP, D), jnp.float32),
    ])

R = 1024


def _dense_body(p_ref, c_ref, h_ref, wl_ref, bl_ref, wr_ref, o_ref, *, relu):
  cnt = c_ref[0][:, 0:1] + c_ref[1][:, 0:1]
  inv = 1.0 / jnp.maximum(cnt, 1.0)
  agg = (p_ref[0] + p_ref[1]) * inv
  dn = (((1,), (1,)), ((), ()))
  acc = lax.dot_general(agg, wl_ref[...], dn,
                        precision=lax.Precision.HIGHEST,
                        preferred_element_type=jnp.float32)
  acc += lax.dot_general(h_ref[...], wr_ref[...], dn,
                         precision=lax.Precision.HIGHEST,
                         preferred_element_type=jnp.float32)
  acc += bl_ref[...]
  o_ref[...] = jnp.maximum(acc, 0.0) if relu else acc


def _final_body(p_ref, c_ref, h_ref, wl_ref, bl_ref, wr_ref, wo_ref, bo_ref,
                o_ref):
  cnt = c_ref[0][:, 0:1] + c_ref[1][:, 0:1]
  inv = 1.0 / jnp.maximum(cnt, 1.0)
  agg = (p_ref[0] + p_ref[1]) * inv
  dn = (((1,), (1,)), ((), ()))
  acc = lax.dot_general(agg, wl_ref[...], dn,
                        precision=lax.Precision.HIGHEST,
                        preferred_element_type=jnp.float32)
  acc += lax.dot_general(h_ref[...], wr_ref[...], dn,
                         precision=lax.Precision.HIGHEST,
                         preferred_element_type=jnp.float32)
  acc += bl_ref[...]
  o_ref[...] = lax.dot_general(acc, wo_ref[...], dn,
                               precision=lax.Precision.HIGHEST,
                               preferred_element_type=jnp.float32) + bo_ref[...]


_W_SPEC = pl.BlockSpec((D, D), lambda i: (0, 0))
_B_SPEC = pl.BlockSpec((1, D), lambda i: (0, 0))
_P_SPEC = pl.BlockSpec((NC, R, D), lambda i: (0, i, 0))
_C_SPEC = pl.BlockSpec((NC, R, D), lambda i: (0, i, 0))
_H_SPEC = pl.BlockSpec((R, D), lambda i: (i, 0))


def _dense_layer(P, C, h, Wl, bl, Wr, relu):
  return pl.pallas_call(
      functools.partial(_dense_body, relu=relu),
      grid=(NP // R,),
      in_specs=[_P_SPEC, _C_SPEC, _H_SPEC, _W_SPEC, _B_SPEC, _W_SPEC],
      out_specs=_H_SPEC,
      out_shape=jax.ShapeDtypeStruct((NP, D), jnp.float32),
  )(P, C, h, Wl, bl, Wr)


def _final_layer(P, C, h, Wl, bl, Wr, Wo, bo):
  return pl.pallas_call(
      _final_body,
      grid=(NP // R,),
      in_specs=[_P_SPEC, _C_SPEC, _H_SPEC, _W_SPEC, _B_SPEC, _W_SPEC,
                pl.BlockSpec((OUT, D), lambda i: (0, 0)),
                pl.BlockSpec((1, OUT), lambda i: (0, 0))],
      out_specs=pl.BlockSpec((R, OUT), lambda i: (i, 0)),
      out_shape=jax.ShapeDtypeStruct((NP, OUT), jnp.float32),
  )(P, C, h, Wl, bl, Wr, Wo, bo)


@jax.jit
def kernel(x, edge_index, Wl0, bl0, Wr0, Wl1, bl1, Wr1, Wl2, bl2, Wr2, Wo, bo):
  srcw = edge_index[0].reshape(NW, CE)
  dstw = edge_index[1].reshape(NW, CE)
  pad_src = jnp.zeros((NW, EP - CE), jnp.int32)
  pad_dst = jnp.broadcast_to(
      (NP - NW + jnp.arange(NW, dtype=jnp.int32))[:, None], (NW, EP - CE))
  src2 = jnp.concatenate([srcw, pad_src], axis=1).reshape(NW, K, B)
  dst2 = jnp.concatenate([dstw, pad_dst], axis=1).reshape(NW, K, B)
  dst2c = edge_index[1].reshape(NW, KC, BC)
  zrows = jnp.zeros((RPT, D), jnp.float32)
  ones = jnp.ones((BC, D), jnp.float32)
  bl0r, bl1r, bl2r = bl0.reshape(1, D), bl1.reshape(1, D), bl2.reshape(1, D)
  bor = bo.reshape(1, OUT)
  xp = jnp.concatenate([x, jnp.zeros((NP - N, D), jnp.float32)], axis=0)

  (C,) = _sc_counts(dst2c, zrows, ones)
  (P0,) = _sc_seg(xp, src2, dst2)
  h1 = _dense_layer(P0, C, xp, Wl0, bl0r, Wr0, True)
  (P1,) = _sc_seg(h1, src2, dst2)
  h2 = _dense_layer(P1, C, h1, Wl1, bl1r, Wr1, True)
  (P2,) = _sc_seg(h2, src2, dst2)
  return _final_layer(P2, C, h2, Wl2, bl2r, Wr2, Wo, bor)[:N]


# speedup vs baseline: 1.1115x; 1.1115x over previous
import functools

import jax
import jax.numpy as jnp
from jax import lax
from jax.experimental import pallas as pl
from jax.experimental.pallas import tpu as pltpu
from jax.experimental.pallas import tpu_sc as plsc

N = 10000
E = 320000
D = 128
OUT = 2

NC = 2
NS = 16
NW = NC * NS

CE = E // NW
B = 80
EP = 10240
K = EP // B
KP = K // 2
BC = 80
KC = CE // BC
NP = 10240
RPT = NP // NS
CPT = NP // NS


_MESH = plsc.VectorSubcoreMesh(core_axis_name="c", subcore_axis_name="s")


def _seg_body(h_hbm, src2, dst2, p_out, idx_s, idx_d, rows_a, rows_b,
              acc, sem_a, sem_b):
  cid = lax.axis_index("c")
  sid = lax.axis_index("s")
  wid = cid * NS + sid

  z16 = jnp.zeros((16,), jnp.float32)

  def zstore(i, _):
    rows_a[i // 8, pl.ds((i % 8) * 16, 16)] = z16
    return _

  lax.fori_loop(0, B * 8, zstore, None)
  for q in range(RPT // B):
    pltpu.sync_copy(rows_a, acc.at[pl.ds(sid * RPT + q * B, B), :])
  plsc.subcore_barrier()

  def gather(c, buf, sem):
    return pltpu.async_copy(h_hbm.at[idx_s.at[c]], buf, sem)

  def gwait(c, buf, sem):
    pltpu.make_async_copy(h_hbm.at[idx_s.at[c]], buf, sem).wait()

  def scatter(c, buf):
    pltpu.sync_copy(buf, acc.at[idx_d.at[c]], add=True)

  def pair(t, _):
    i = 2 * t
    gather(i + 1, rows_b, sem_b)
    gwait(i, rows_a, sem_a)
    scatter(i, rows_a)
    gather(i + 2, rows_a, sem_a)
    gwait(i + 1, rows_b, sem_b)
    scatter(i + 1, rows_b)
    return _

  for p in range(K // KP):
    pltpu.sync_copy(src2.at[wid, pl.ds(p * KP, KP), :], idx_s)
    pltpu.sync_copy(dst2.at[wid, pl.ds(p * KP, KP), :], idx_d)
    gather(0, rows_a, sem_a)
    lax.fori_loop(0, KP // 2 - 1, pair, None)
    gather(KP - 1, rows_b, sem_b)
    gwait(KP - 2, rows_a, sem_a)
    scatter(KP - 2, rows_a)
    gwait(KP - 1, rows_b, sem_b)
    scatter(KP - 1, rows_b)

  plsc.subcore_barrier()
  pltpu.sync_copy(acc.at[pl.ds(sid * RPT, RPT), :],
                  p_out.at[cid, pl.ds(sid * RPT, RPT), :])


_sc_seg = pl.kernel(
    _seg_body,
    out_type=(jax.ShapeDtypeStruct((NC, NP, D), jnp.float32),),
    mesh=_MESH,
    scratch_types=[
        pltpu.VMEM((KP, B), jnp.int32),
        pltpu.VMEM((KP, B), jnp.int32),
        pltpu.VMEM((B, D), jnp.float32),
        pltpu.VMEM((B, D), jnp.float32),
        pltpu.VMEM_SHARED((NP, D), jnp.float32),
        pltpu.SemaphoreType.DMA,
        pltpu.SemaphoreType.DMA,
    ])


def _cnt_body(dst2, zrows, ones_hbm, c_out, idx_d, ones_v, cacc):
  cid = lax.axis_index("c")
  sid = lax.axis_index("s")
  wid = cid * NS + sid

  pltpu.sync_copy(dst2.at[wid], idx_d)
  pltpu.sync_copy(zrows, cacc.at[pl.ds(sid * RPT, RPT), :])
  pltpu.sync_copy(ones_hbm, ones_v)
  plsc.subcore_barrier()

  def chunk(c, _):
    pltpu.sync_copy(ones_v, cacc.at[idx_d.at[c]], add=True)
    return _

  lax.fori_loop(0, KC, chunk, None)
  plsc.subcore_barrier()

  pltpu.sync_copy(cacc.at[pl.ds(sid * RPT, RPT), :],
                  c_out.at[cid, pl.ds(sid * RPT, RPT), :])


_sc_counts = pl.kernel(
    _cnt_body,
    out_type=(jax.ShapeDtypeStruct((NC, NP, D), jnp.float32),),
    mesh=_MESH,
    scratch_types=[
        pltpu.VMEM((KC, BC), jnp.int32),
        pltpu.VMEM((BC, D), jnp.float32),
        pltpu.VMEM_SHARED((NP, D), jnp.float32),
    ])

R = 1024


def _dense_body(p_ref, c_ref, h_ref, wl_ref, bl_ref, wr_ref, o_ref, *, relu):
  cnt = c_ref[0][:, 0:1] + c_ref[1][:, 0:1]
  inv = 1.0 / jnp.maximum(cnt, 1.0)
  agg = (p_ref[0] + p_ref[1]) * inv
  dn = (((1,), (1,)), ((), ()))
  acc = lax.dot_general(agg, wl_ref[...], dn,
                        precision=lax.Precision.HIGHEST,
                        preferred_element_type=jnp.float32)
  acc += lax.dot_general(h_ref[...], wr_ref[...], dn,
                         precision=lax.Precision.HIGHEST,
                         preferred_element_type=jnp.float32)
  acc += bl_ref[...]
  o_ref[...] = jnp.maximum(acc, 0.0) if relu else acc


def _final_body(p_ref, c_ref, h_ref, wl_ref, bl_ref, wr_ref, wo_ref, bo_ref,
                o_ref):
  cnt = c_ref[0][:, 0:1] + c_ref[1][:, 0:1]
  inv = 1.0 / jnp.maximum(cnt, 1.0)
  agg = (p_ref[0] + p_ref[1]) * inv
  dn = (((1,), (1,)), ((), ()))
  acc = lax.dot_general(agg, wl_ref[...], dn,
                        precision=lax.Precision.HIGHEST,
                        preferred_element_type=jnp.float32)
  acc += lax.dot_general(h_ref[...], wr_ref[...], dn,
                         precision=lax.Precision.HIGHEST,
                         preferred_element_type=jnp.float32)
  acc += bl_ref[...]
  o_ref[...] = lax.dot_general(acc, wo_ref[...], dn,
                               precision=lax.Precision.HIGHEST,
                               preferred_element_type=jnp.float32) + bo_ref[...]


_W_SPEC = pl.BlockSpec((D, D), lambda i: (0, 0))
_B_SPEC = pl.BlockSpec((1, D), lambda i: (0, 0))
_P_SPEC = pl.BlockSpec((NC, R, D), lambda i: (0, i, 0))
_C_SPEC = pl.BlockSpec((NC, R, D), lambda i: (0, i, 0))
_H_SPEC = pl.BlockSpec((R, D), lambda i: (i, 0))


def _dense_layer(P, C, h, Wl, bl, Wr, relu):
  return pl.pallas_call(
      functools.partial(_dense_body, relu=relu),
      grid=(NP // R,),
      in_specs=[_P_SPEC, _C_SPEC, _H_SPEC, _W_SPEC, _B_SPEC, _W_SPEC],
      out_specs=_H_SPEC,
      out_shape=jax.ShapeDtypeStruct((NP, D), jnp.float32),
  )(P, C, h, Wl, bl, Wr)


def _final_layer(P, C, h, Wl, bl, Wr, Wo, bo):
  return pl.pallas_call(
      _final_body,
      grid=(NP // R,),
      in_specs=[_P_SPEC, _C_SPEC, _H_SPEC, _W_SPEC, _B_SPEC, _W_SPEC,
                pl.BlockSpec((OUT, D), lambda i: (0, 0)),
                pl.BlockSpec((1, OUT), lambda i: (0, 0))],
      out_specs=pl.BlockSpec((R, OUT), lambda i: (i, 0)),
      out_shape=jax.ShapeDtypeStruct((NP, OUT), jnp.float32),
  )(P, C, h, Wl, bl, Wr, Wo, bo)


@jax.jit
def kernel(x, edge_index, Wl0, bl0, Wr0, Wl1, bl1, Wr1, Wl2, bl2, Wr2, Wo, bo):
  srcw = edge_index[0].reshape(NW, CE)
  dstw = edge_index[1].reshape(NW, CE)
  pad_src = jnp.zeros((NW, EP - CE), jnp.int32)
  pad_dst = jnp.broadcast_to(
      (NP - NW + jnp.arange(NW, dtype=jnp.int32))[:, None], (NW, EP - CE))
  src2 = jnp.concatenate([srcw, pad_src], axis=1).reshape(NW, K, B)
  dst2 = jnp.concatenate([dstw, pad_dst], axis=1).reshape(NW, K, B)
  dst2c = edge_index[1].reshape(NW, KC, BC)
  zrows = jnp.zeros((RPT, D), jnp.float32)
  ones = jnp.ones((BC, D), jnp.float32)
  bl0r, bl1r, bl2r = bl0.reshape(1, D), bl1.reshape(1, D), bl2.reshape(1, D)
  bor = bo.reshape(1, OUT)
  xp = jnp.concatenate([x, jnp.zeros((NP - N, D), jnp.float32)], axis=0)

  (C,) = _sc_counts(dst2c, zrows, ones)
  (P0,) = _sc_seg(xp, src2, dst2)
  h1 = _dense_layer(P0, C, xp, Wl0, bl0r, Wr0, True)
  (P1,) = _sc_seg(h1, src2, dst2)
  h2 = _dense_layer(P1, C, h1, Wl1, bl1r, Wr1, True)
  (P2,) = _sc_seg(h2, src2, dst2)
  return _final_layer(P2, C, h2, Wl2, bl2r, Wr2, Wo, bor)[:N]


# speedup vs baseline: 2.8311x; 2.5471x over previous
import functools

import jax
import jax.numpy as jnp
from jax import lax
from jax.experimental import pallas as pl
from jax.experimental.pallas import tpu as pltpu
from jax.experimental.pallas import tpu_sc as plsc

N = 10000
E = 320000
D = 128
OUT = 2

NC = 2
NS = 16
NW = NC * NS

CE = E // NW
B = 80
EP = 10240
K = EP // B
KP = K // 2
BC = 80
KC = CE // BC
NP = 10240
RPT = NP // NS
CPT = NP // NS


_MESH = plsc.VectorSubcoreMesh(core_axis_name="c", subcore_axis_name="s")


def _seg_body(h_hbm, src2, dst2, p_out, idx_s, idx_d, rows_a, rows_b,
              acc, sem_a, sem_b):
  cid = lax.axis_index("c")
  sid = lax.axis_index("s")
  wid = cid * NS + sid

  z16 = jnp.zeros((16,), jnp.float32)

  def zstore(i, _):
    rows_a[i // 8, pl.ds((i % 8) * 16, 16)] = z16
    return _

  lax.fori_loop(0, B * 8, zstore, None)
  for q in range(RPT // B):
    pltpu.sync_copy(rows_a, acc.at[pl.ds(sid * RPT + q * B, B), :])
  plsc.subcore_barrier()

  def gather(c, buf, sem):
    return pltpu.async_copy(h_hbm.at[idx_s.at[c]], buf, sem)

  def gwait(c, buf, sem):
    pltpu.make_async_copy(h_hbm.at[idx_s.at[c]], buf, sem).wait()

  def scatter(c, buf):
    pltpu.sync_copy(buf, acc.at[idx_d.at[c]], add=True)

  def pair(t, _):
    i = 2 * t
    gather(i + 1, rows_b, sem_b)
    gwait(i, rows_a, sem_a)
    scatter(i, rows_a)
    gather(i + 2, rows_a, sem_a)
    gwait(i + 1, rows_b, sem_b)
    scatter(i + 1, rows_b)
    return _

  for p in range(K // KP):
    pltpu.sync_copy(src2.at[wid, pl.ds(p * KP, KP), :], idx_s)
    pltpu.sync_copy(dst2.at[wid, pl.ds(p * KP, KP), :], idx_d)
    gather(0, rows_a, sem_a)
    lax.fori_loop(0, KP // 2 - 1, pair, None)
    gather(KP - 1, rows_b, sem_b)
    gwait(KP - 2, rows_a, sem_a)
    scatter(KP - 2, rows_a)
    gwait(KP - 1, rows_b, sem_b)
    scatter(KP - 1, rows_b)

  plsc.subcore_barrier()
  pltpu.sync_copy(acc.at[pl.ds(sid * RPT, RPT), :],
                  p_out.at[cid, pl.ds(sid * RPT, RPT), :])


_sc_seg = pl.kernel(
    _seg_body,
    out_type=(jax.ShapeDtypeStruct((NC, NP, D), jnp.float32),),
    mesh=_MESH,
    scratch_types=[
        pltpu.VMEM((KP, B), jnp.int32),
        pltpu.VMEM((KP, B), jnp.int32),
        pltpu.VMEM((B, D), jnp.float32),
        pltpu.VMEM((B, D), jnp.float32),
        pltpu.VMEM_SHARED((NP, D), jnp.float32),
        pltpu.SemaphoreType.DMA,
        pltpu.SemaphoreType.DMA,
    ])


def _cnt_body(dst2, zrows, ones_hbm, c_out, idx_d, ones_v, cacc):
  cid = lax.axis_index("c")
  sid = lax.axis_index("s")
  wid = cid * NS + sid

  pltpu.sync_copy(dst2.at[wid], idx_d)
  pltpu.sync_copy(zrows, cacc.at[pl.ds(sid * RPT, RPT), :])
  pltpu.sync_copy(ones_hbm, ones_v)
  plsc.subcore_barrier()

  def chunk(c, _):
    pltpu.sync_copy(ones_v, cacc.at[idx_d.at[c]], add=True)
    return _

  lax.fori_loop(0, KC, chunk, None)
  plsc.subcore_barrier()

  pltpu.sync_copy(cacc.at[pl.ds(sid * RPT, RPT), :],
                  c_out.at[cid, pl.ds(sid * RPT, RPT), :])


_sc_counts = pl.kernel(
    _cnt_body,
    out_type=(jax.ShapeDtypeStruct((NC, NP, D), jnp.float32),),
    mesh=_MESH,
    scratch_types=[
        pltpu.VMEM((KC, BC), jnp.int32),
        pltpu.VMEM((BC, D), jnp.float32),
        pltpu.VMEM_SHARED((NP, D), jnp.float32),
    ])

R = 1024


def _dense_body(p_ref, c_ref, h_ref, wl_ref, bl_ref, wr_ref, o_ref, *, relu):
  cnt = c_ref[0][:, 0:1] + c_ref[1][:, 0:1]
  inv = 1.0 / jnp.maximum(cnt, 1.0)
  agg = (p_ref[0] + p_ref[1]) * inv
  dn = (((1,), (1,)), ((), ()))
  acc = lax.dot_general(agg, wl_ref[...], dn,
                        precision=lax.Precision.HIGHEST,
                        preferred_element_type=jnp.float32)
  acc += lax.dot_general(h_ref[...], wr_ref[...], dn,
                         precision=lax.Precision.HIGHEST,
                         preferred_element_type=jnp.float32)
  acc += bl_ref[...]
  o_ref[...] = jnp.maximum(acc, 0.0) if relu else acc


def _final_body(p_ref, c_ref, h_ref, wl_ref, bl_ref, wr_ref, wo_ref, bo_ref,
                o_ref):
  cnt = c_ref[0][:, 0:1] + c_ref[1][:, 0:1]
  inv = 1.0 / jnp.maximum(cnt, 1.0)
  agg = (p_ref[0] + p_ref[1]) * inv
  dn = (((1,), (1,)), ((), ()))
  acc = lax.dot_general(agg, wl_ref[...], dn,
                        precision=lax.Precision.HIGHEST,
                        preferred_element_type=jnp.float32)
  acc += lax.dot_general(h_ref[...], wr_ref[...], dn,
                         precision=lax.Precision.HIGHEST,
                         preferred_element_type=jnp.float32)
  acc += bl_ref[...]
  o_ref[...] = lax.dot_general(acc, wo_ref[...], dn,
                               precision=lax.Precision.HIGHEST,
                               preferred_element_type=jnp.float32) + bo_ref[...]


_W_SPEC = pl.BlockSpec((D, D), lambda i: (0, 0))
_B_SPEC = pl.BlockSpec((1, D), lambda i: (0, 0))
_P_SPEC = pl.BlockSpec((NC, R, D), lambda i: (0, i, 0))
_C_SPEC = pl.BlockSpec((NC, R, D), lambda i: (0, i, 0))
_H_SPEC = pl.BlockSpec((R, D), lambda i: (i, 0))


def _dense_layer(P, C, h, Wl, bl, Wr, relu):
  return pl.pallas_call(
      functools.partial(_dense_body, relu=relu),
      grid=(NP // R,),
      in_specs=[_P_SPEC, _C_SPEC, _H_SPEC, _W_SPEC, _B_SPEC, _W_SPEC],
      out_specs=_H_SPEC,
      out_shape=jax.ShapeDtypeStruct((NP, D), jnp.float32),
  )(P, C, h, Wl, bl, Wr)


def _final_layer(P, C, h, Wl, bl, Wr, Wo, bo):
  return pl.pallas_call(
      _final_body,
      grid=(NP // R,),
      in_specs=[_P_SPEC, _C_SPEC, _H_SPEC, _W_SPEC, _B_SPEC, _W_SPEC,
                pl.BlockSpec((OUT, D), lambda i: (0, 0)),
                pl.BlockSpec((1, OUT), lambda i: (0, 0))],
      out_specs=pl.BlockSpec((R, OUT), lambda i: (i, 0)),
      out_shape=jax.ShapeDtypeStruct((NP, OUT), jnp.float32),
  )(P, C, h, Wl, bl, Wr, Wo, bo)


@jax.jit
def kernel(x, edge_index, Wl0, bl0, Wr0, Wl1, bl1, Wr1, Wl2, bl2, Wr2, Wo, bo):
  srcw = edge_index[0].reshape(NW, CE)
  dstw = edge_index[1].reshape(NW, CE)
  pad_iota = jnp.arange(EP - CE, dtype=jnp.int32)
  pad_src = jnp.broadcast_to(pad_iota[None, :] * 41 % N, (NW, EP - CE))
  pad_dst = jnp.broadcast_to(N + pad_iota[None, :], (NW, EP - CE))
  src2 = jnp.concatenate([srcw, pad_src], axis=1).reshape(NW, K, B)
  dst2 = jnp.concatenate([dstw, pad_dst], axis=1).reshape(NW, K, B)
  dst2c = edge_index[1].reshape(NW, KC, BC)
  zrows = jnp.zeros((RPT, D), jnp.float32)
  ones = jnp.ones((BC, D), jnp.float32)
  bl0r, bl1r, bl2r = bl0.reshape(1, D), bl1.reshape(1, D), bl2.reshape(1, D)
  bor = bo.reshape(1, OUT)
  xp = jnp.concatenate([x, jnp.zeros((NP - N, D), jnp.float32)], axis=0)

  (C,) = _sc_counts(dst2c, zrows, ones)
  (P0,) = _sc_seg(xp, src2, dst2)
  h1 = _dense_layer(P0, C, xp, Wl0, bl0r, Wr0, True)
  (P1,) = _sc_seg(h1, src2, dst2)
  h2 = _dense_layer(P1, C, h1, Wl1, bl1r, Wr1, True)
  (P2,) = _sc_seg(h2, src2, dst2)
  return _final_layer(P2, C, h2, Wl2, bl2r, Wr2, Wo, bor)[:N]


# speedup vs baseline: 2.8373x; 1.0022x over previous
import functools

import jax
import jax.numpy as jnp
from jax import lax
from jax.experimental import pallas as pl
from jax.experimental.pallas import tpu as pltpu
from jax.experimental.pallas import tpu_sc as plsc

N = 10000
E = 320000
D = 128
OUT = 2

NC = 2
NS = 16
NW = NC * NS

CE = E // NW
B = 80
EP = 10240
K = EP // B
KP = K // 2
BC = 80
KC = CE // BC
NP = 10240
RPT = NP // NS
CPT = NP // NS


_MESH = plsc.VectorSubcoreMesh(core_axis_name="c", subcore_axis_name="s")


def _seg_body(h_hbm, src2, dst2, p_out, idx_s, idx_d, rows_a, rows_b,
              acc, sem_a, sem_b):
  cid = lax.axis_index("c")
  sid = lax.axis_index("s")
  wid = cid * NS + sid

  z16 = jnp.zeros((16,), jnp.float32)

  def zstore(i, _):
    rows_a[i // 8, pl.ds((i % 8) * 16, 16)] = z16
    return _

  lax.fori_loop(0, B * 8, zstore, None)
  for q in range(RPT // B):
    pltpu.sync_copy(rows_a, acc.at[pl.ds(sid * RPT + q * B, B), :])
  plsc.subcore_barrier()

  def gather(c, buf, sem):
    return pltpu.async_copy(h_hbm.at[idx_s.at[c]], buf, sem)

  def gwait(c, buf, sem):
    pltpu.make_async_copy(h_hbm.at[idx_s.at[c]], buf, sem).wait()

  def scatter(c, buf):
    pltpu.sync_copy(buf, acc.at[idx_d.at[c]], add=True)

  def pair(t, _):
    i = 2 * t
    gather(i + 1, rows_b, sem_b)
    gwait(i, rows_a, sem_a)
    scatter(i, rows_a)
    gather(i + 2, rows_a, sem_a)
    gwait(i + 1, rows_b, sem_b)
    scatter(i + 1, rows_b)
    return _

  for p in range(K // KP):
    pltpu.sync_copy(src2.at[wid, pl.ds(p * KP, KP), :], idx_s)
    pltpu.sync_copy(dst2.at[wid, pl.ds(p * KP, KP), :], idx_d)
    gather(0, rows_a, sem_a)
    lax.fori_loop(0, KP // 2 - 1, pair, None)
    gather(KP - 1, rows_b, sem_b)
    gwait(KP - 2, rows_a, sem_a)
    scatter(KP - 2, rows_a)
    gwait(KP - 1, rows_b, sem_b)
    scatter(KP - 1, rows_b)

  plsc.subcore_barrier()
  pltpu.sync_copy(acc.at[pl.ds(sid * RPT, RPT), :],
                  p_out.at[cid, pl.ds(sid * RPT, RPT), :])


_sc_seg = pl.kernel(
    _seg_body,
    out_type=(jax.ShapeDtypeStruct((NC, NP, D), jnp.float32),),
    mesh=_MESH,
    scratch_types=[
        pltpu.VMEM((KP, B), jnp.int32),
        pltpu.VMEM((KP, B), jnp.int32),
        pltpu.VMEM((B, D), jnp.float32),
        pltpu.VMEM((B, D), jnp.float32),
        pltpu.VMEM_SHARED((NP, D), jnp.float32),
        pltpu.SemaphoreType.DMA,
        pltpu.SemaphoreType.DMA,
    ])


WNC = 8


def _cnt_body(dst2, zrows, ones_hbm, c_out, idx_d, ones_v, cacc, sem):
  cid = lax.axis_index("c")
  sid = lax.axis_index("s")
  wid = cid * NS + sid

  pltpu.sync_copy(dst2.at[wid], idx_d)
  pltpu.sync_copy(zrows, cacc.at[pl.ds(sid * RPT, RPT), :])
  pltpu.sync_copy(ones_hbm, ones_v)
  plsc.subcore_barrier()

  def fire(c, _):
    pltpu.async_copy(ones_v, cacc.at[idx_d.at[c]], sem, add=True)
    return _

  def fire_drain(c, _):
    pltpu.async_copy(ones_v, cacc.at[idx_d.at[c + WNC]], sem, add=True)
    pltpu.make_async_copy(ones_v, cacc.at[idx_d.at[c]], sem).wait()
    return _

  def drain(c, _):
    pltpu.make_async_copy(ones_v, cacc.at[idx_d.at[c]], sem).wait()
    return _

  lax.fori_loop(0, WNC, fire, None)
  lax.fori_loop(0, KC - WNC, fire_drain, None)
  lax.fori_loop(KC - WNC, KC, drain, None)
  plsc.subcore_barrier()

  pltpu.sync_copy(cacc.at[pl.ds(sid * RPT, RPT), :],
                  c_out.at[cid, pl.ds(sid * RPT, RPT), :])


_sc_counts = pl.kernel(
    _cnt_body,
    out_type=(jax.ShapeDtypeStruct((NC, NP, D), jnp.float32),),
    mesh=_MESH,
    scratch_types=[
        pltpu.VMEM((KC, BC), jnp.int32),
        pltpu.VMEM((BC, D), jnp.float32),
        pltpu.VMEM_SHARED((NP, D), jnp.float32),
        pltpu.SemaphoreType.DMA,
    ])

R = 1024


def _dense_body(p_ref, c_ref, h_ref, wl_ref, bl_ref, wr_ref, o_ref, *, relu):
  cnt = c_ref[0][:, 0:1] + c_ref[1][:, 0:1]
  inv = 1.0 / jnp.maximum(cnt, 1.0)
  agg = (p_ref[0] + p_ref[1]) * inv
  dn = (((1,), (1,)), ((), ()))
  acc = lax.dot_general(agg, wl_ref[...], dn,
                        precision=lax.Precision.HIGHEST,
                        preferred_element_type=jnp.float32)
  acc += lax.dot_general(h_ref[...], wr_ref[...], dn,
                         precision=lax.Precision.HIGHEST,
                         preferred_element_type=jnp.float32)
  acc += bl_ref[...]
  o_ref[...] = jnp.maximum(acc, 0.0) if relu else acc


def _final_body(p_ref, c_ref, h_ref, wl_ref, bl_ref, wr_ref, wo_ref, bo_ref,
                o_ref):
  cnt = c_ref[0][:, 0:1] + c_ref[1][:, 0:1]
  inv = 1.0 / jnp.maximum(cnt, 1.0)
  agg = (p_ref[0] + p_ref[1]) * inv
  dn = (((1,), (1,)), ((), ()))
  acc = lax.dot_general(agg, wl_ref[...], dn,
                        precision=lax.Precision.HIGHEST,
                        preferred_element_type=jnp.float32)
  acc += lax.dot_general(h_ref[...], wr_ref[...], dn,
                         precision=lax.Precision.HIGHEST,
                         preferred_element_type=jnp.float32)
  acc += bl_ref[...]
  o_ref[...] = lax.dot_general(acc, wo_ref[...], dn,
                               precision=lax.Precision.HIGHEST,
                               preferred_element_type=jnp.float32) + bo_ref[...]


_W_SPEC = pl.BlockSpec((D, D), lambda i: (0, 0))
_B_SPEC = pl.BlockSpec((1, D), lambda i: (0, 0))
_P_SPEC = pl.BlockSpec((NC, R, D), lambda i: (0, i, 0))
_C_SPEC = pl.BlockSpec((NC, R, D), lambda i: (0, i, 0))
_H_SPEC = pl.BlockSpec((R, D), lambda i: (i, 0))


def _dense_layer(P, C, h, Wl, bl, Wr, relu):
  return pl.pallas_call(
      functools.partial(_dense_body, relu=relu),
      grid=(NP // R,),
      in_specs=[_P_SPEC, _C_SPEC, _H_SPEC, _W_SPEC, _B_SPEC, _W_SPEC],
      out_specs=_H_SPEC,
      out_shape=jax.ShapeDtypeStruct((NP, D), jnp.float32),
  )(P, C, h, Wl, bl, Wr)


def _final_layer(P, C, h, Wl, bl, Wr, Wo, bo):
  return pl.pallas_call(
      _final_body,
      grid=(NP // R,),
      in_specs=[_P_SPEC, _C_SPEC, _H_SPEC, _W_SPEC, _B_SPEC, _W_SPEC,
                pl.BlockSpec((OUT, D), lambda i: (0, 0)),
                pl.BlockSpec((1, OUT), lambda i: (0, 0))],
      out_specs=pl.BlockSpec((R, OUT), lambda i: (i, 0)),
      out_shape=jax.ShapeDtypeStruct((NP, OUT), jnp.float32),
  )(P, C, h, Wl, bl, Wr, Wo, bo)


@jax.jit
def kernel(x, edge_index, Wl0, bl0, Wr0, Wl1, bl1, Wr1, Wl2, bl2, Wr2, Wo, bo):
  srcw = edge_index[0].reshape(NW, CE)
  dstw = edge_index[1].reshape(NW, CE)
  pad_iota = jnp.arange(EP - CE, dtype=jnp.int32)
  pad_src = jnp.broadcast_to(pad_iota[None, :] * 41 % N, (NW, EP - CE))
  pad_dst = jnp.broadcast_to(N + pad_iota[None, :], (NW, EP - CE))
  src2 = jnp.concatenate([srcw, pad_src], axis=1).reshape(NW, K, B)
  dst2 = jnp.concatenate([dstw, pad_dst], axis=1).reshape(NW, K, B)
  dst2c = edge_index[1].reshape(NW, KC, BC)
  zrows = jnp.zeros((RPT, D), jnp.float32)
  ones = jnp.ones((BC, D), jnp.float32)
  bl0r, bl1r, bl2r = bl0.reshape(1, D), bl1.reshape(1, D), bl2.reshape(1, D)
  bor = bo.reshape(1, OUT)
  xp = jnp.concatenate([x, jnp.zeros((NP - N, D), jnp.float32)], axis=0)

  (C,) = _sc_counts(dst2c, zrows, ones)
  (P0,) = _sc_seg(xp, src2, dst2)
  h1 = _dense_layer(P0, C, xp, Wl0, bl0r, Wr0, True)
  (P1,) = _sc_seg(h1, src2, dst2)
  h2 = _dense_layer(P1, C, h1, Wl1, bl1r, Wr1, True)
  (P2,) = _sc_seg(h2, src2, dst2)
  return _final_layer(P2, C, h2, Wl2, bl2r, Wr2, Wo, bor)[:N]


# speedup vs baseline: 2.8388x; 1.0005x over previous
import functools

import jax
import jax.numpy as jnp
from jax import lax
from jax.experimental import pallas as pl
from jax.experimental.pallas import tpu as pltpu
from jax.experimental.pallas import tpu_sc as plsc

N = 10000
E = 320000
D = 128
OUT = 2

NC = 2
NS = 16
NW = NC * NS

CE = E // NW
B = 80
EP = 10240
K = EP // B
KP = K // 2
BC = 80
KC = CE // BC
NP = 10240
RPT = NP // NS
CPT = NP // NS


_MESH = plsc.VectorSubcoreMesh(core_axis_name="c", subcore_axis_name="s")


def _seg_body(h_hbm, src2, dst2, p_out, idx_s, idx_d, rows_a, rows_b,
              acc, sem_a, sem_b):
  cid = lax.axis_index("c")
  sid = lax.axis_index("s")
  wid = cid * NS + sid

  z16 = jnp.zeros((16,), jnp.float32)

  def zstore(i, _):
    rows_a[i // 8, pl.ds((i % 8) * 16, 16)] = z16
    return _

  lax.fori_loop(0, B * 8, zstore, None)
  for q in range(RPT // B):
    pltpu.sync_copy(rows_a, acc.at[pl.ds(sid * RPT + q * B, B), :])
  plsc.subcore_barrier()

  HB = B // 2

  def gather(c, buf, sem):
    pltpu.async_copy(h_hbm.at[idx_s.at[c, pl.ds(0, HB)]],
                     buf.at[pl.ds(0, HB), :], sem)
    pltpu.async_copy(h_hbm.at[idx_s.at[c, pl.ds(HB, HB)]],
                     buf.at[pl.ds(HB, HB), :], sem)

  def gwait(c, buf, sem):
    pltpu.make_async_copy(h_hbm.at[idx_s.at[c, pl.ds(0, HB)]],
                          buf.at[pl.ds(0, HB), :], sem).wait()
    pltpu.make_async_copy(h_hbm.at[idx_s.at[c, pl.ds(HB, HB)]],
                          buf.at[pl.ds(HB, HB), :], sem).wait()

  def scatter(c, buf):
    pltpu.sync_copy(buf, acc.at[idx_d.at[c]], add=True)

  def pair(t, _):
    i = 2 * t
    gather(i + 1, rows_b, sem_b)
    gwait(i, rows_a, sem_a)
    scatter(i, rows_a)
    gather(i + 2, rows_a, sem_a)
    gwait(i + 1, rows_b, sem_b)
    scatter(i + 1, rows_b)
    return _

  for p in range(K // KP):
    pltpu.sync_copy(src2.at[wid, pl.ds(p * KP, KP), :], idx_s)
    pltpu.sync_copy(dst2.at[wid, pl.ds(p * KP, KP), :], idx_d)
    gather(0, rows_a, sem_a)
    lax.fori_loop(0, KP // 2 - 1, pair, None)
    gather(KP - 1, rows_b, sem_b)
    gwait(KP - 2, rows_a, sem_a)
    scatter(KP - 2, rows_a)
    gwait(KP - 1, rows_b, sem_b)
    scatter(KP - 1, rows_b)

  plsc.subcore_barrier()
  pltpu.sync_copy(acc.at[pl.ds(sid * RPT, RPT), :],
                  p_out.at[cid, pl.ds(sid * RPT, RPT), :])


_sc_seg = pl.kernel(
    _seg_body,
    out_type=(jax.ShapeDtypeStruct((NC, NP, D), jnp.float32),),
    mesh=_MESH,
    scratch_types=[
        pltpu.VMEM((KP, B), jnp.int32),
        pltpu.VMEM((KP, B), jnp.int32),
        pltpu.VMEM((B, D), jnp.float32),
        pltpu.VMEM((B, D), jnp.float32),
        pltpu.VMEM_SHARED((NP, D), jnp.float32),
        pltpu.SemaphoreType.DMA,
        pltpu.SemaphoreType.DMA,
    ])


WNC = 8


def _cnt_body(dst2, zrows, ones_hbm, c_out, idx_d, ones_v, cacc, sem):
  cid = lax.axis_index("c")
  sid = lax.axis_index("s")
  wid = cid * NS + sid

  pltpu.sync_copy(dst2.at[wid], idx_d)
  pltpu.sync_copy(zrows, cacc.at[pl.ds(sid * RPT, RPT), :])
  pltpu.sync_copy(ones_hbm, ones_v)
  plsc.subcore_barrier()

  def fire(c, _):
    pltpu.async_copy(ones_v, cacc.at[idx_d.at[c]], sem, add=True)
    return _

  def fire_drain(c, _):
    pltpu.async_copy(ones_v, cacc.at[idx_d.at[c + WNC]], sem, add=True)
    pltpu.make_async_copy(ones_v, cacc.at[idx_d.at[c]], sem).wait()
    return _

  def drain(c, _):
    pltpu.make_async_copy(ones_v, cacc.at[idx_d.at[c]], sem).wait()
    return _

  lax.fori_loop(0, WNC, fire, None)
  lax.fori_loop(0, KC - WNC, fire_drain, None)
  lax.fori_loop(KC - WNC, KC, drain, None)
  plsc.subcore_barrier()

  pltpu.sync_copy(cacc.at[pl.ds(sid * RPT, RPT), :],
                  c_out.at[cid, pl.ds(sid * RPT, RPT), :])


_sc_counts = pl.kernel(
    _cnt_body,
    out_type=(jax.ShapeDtypeStruct((NC, NP, D), jnp.float32),),
    mesh=_MESH,
    scratch_types=[
        pltpu.VMEM((KC, BC), jnp.int32),
        pltpu.VMEM((BC, D), jnp.float32),
        pltpu.VMEM_SHARED((NP, D), jnp.float32),
        pltpu.SemaphoreType.DMA,
    ])

R = 1024


def _dense_body(p_ref, c_ref, h_ref, wl_ref, bl_ref, wr_ref, o_ref, *, relu):
  cnt = c_ref[0][:, 0:1] + c_ref[1][:, 0:1]
  inv = 1.0 / jnp.maximum(cnt, 1.0)
  agg = (p_ref[0] + p_ref[1]) * inv
  dn = (((1,), (1,)), ((), ()))
  acc = lax.dot_general(agg, wl_ref[...], dn,
                        precision=lax.Precision.HIGHEST,
                        preferred_element_type=jnp.float32)
  acc += lax.dot_general(h_ref[...], wr_ref[...], dn,
                         precision=lax.Precision.HIGHEST,
                         preferred_element_type=jnp.float32)
  acc += bl_ref[...]
  o_ref[...] = jnp.maximum(acc, 0.0) if relu else acc


def _final_body(p_ref, c_ref, h_ref, wl_ref, bl_ref, wr_ref, wo_ref, bo_ref,
                o_ref):
  cnt = c_ref[0][:, 0:1] + c_ref[1][:, 0:1]
  inv = 1.0 / jnp.maximum(cnt, 1.0)
  agg = (p_ref[0] + p_ref[1]) * inv
  dn = (((1,), (1,)), ((), ()))
  acc = lax.dot_general(agg, wl_ref[...], dn,
                        precision=lax.Precision.HIGHEST,
                        preferred_element_type=jnp.float32)
  acc += lax.dot_general(h_ref[...], wr_ref[...], dn,
                         precision=lax.Precision.HIGHEST,
                         preferred_element_type=jnp.float32)
  acc += bl_ref[...]
  o_ref[...] = lax.dot_general(acc, wo_ref[...], dn,
                               precision=lax.Precision.HIGHEST,
                               preferred_element_type=jnp.float32) + bo_ref[...]


_W_SPEC = pl.BlockSpec((D, D), lambda i: (0, 0))
_B_SPEC = pl.BlockSpec((1, D), lambda i: (0, 0))
_P_SPEC = pl.BlockSpec((NC, R, D), lambda i: (0, i, 0))
_C_SPEC = pl.BlockSpec((NC, R, D), lambda i: (0, i, 0))
_H_SPEC = pl.BlockSpec((R, D), lambda i: (i, 0))


def _dense_layer(P, C, h, Wl, bl, Wr, relu):
  return pl.pallas_call(
      functools.partial(_dense_body, relu=relu),
      grid=(NP // R,),
      in_specs=[_P_SPEC, _C_SPEC, _H_SPEC, _W_SPEC, _B_SPEC, _W_SPEC],
      out_specs=_H_SPEC,
      out_shape=jax.ShapeDtypeStruct((NP, D), jnp.float32),
  )(P, C, h, Wl, bl, Wr)


def _final_layer(P, C, h, Wl, bl, Wr, Wo, bo):
  return pl.pallas_call(
      _final_body,
      grid=(NP // R,),
      in_specs=[_P_SPEC, _C_SPEC, _H_SPEC, _W_SPEC, _B_SPEC, _W_SPEC,
                pl.BlockSpec((OUT, D), lambda i: (0, 0)),
                pl.BlockSpec((1, OUT), lambda i: (0, 0))],
      out_specs=pl.BlockSpec((R, OUT), lambda i: (i, 0)),
      out_shape=jax.ShapeDtypeStruct((NP, OUT), jnp.float32),
  )(P, C, h, Wl, bl, Wr, Wo, bo)


@jax.jit
def kernel(x, edge_index, Wl0, bl0, Wr0, Wl1, bl1, Wr1, Wl2, bl2, Wr2, Wo, bo):
  srcw = edge_index[0].reshape(NW, CE)
  dstw = edge_index[1].reshape(NW, CE)
  pad_iota = jnp.arange(EP - CE, dtype=jnp.int32)
  pad_src = jnp.broadcast_to(pad_iota[None, :] * 41 % N, (NW, EP - CE))
  pad_dst = jnp.broadcast_to(N + pad_iota[None, :], (NW, EP - CE))
  src2 = jnp.concatenate([srcw, pad_src], axis=1).reshape(NW, K, B)
  dst2 = jnp.concatenate([dstw, pad_dst], axis=1).reshape(NW, K, B)
  dst2c = edge_index[1].reshape(NW, KC, BC)
  zrows = jnp.zeros((RPT, D), jnp.float32)
  ones = jnp.ones((BC, D), jnp.float32)
  bl0r, bl1r, bl2r = bl0.reshape(1, D), bl1.reshape(1, D), bl2.reshape(1, D)
  bor = bo.reshape(1, OUT)
  xp = jnp.concatenate([x, jnp.zeros((NP - N, D), jnp.float32)], axis=0)

  (C,) = _sc_counts(dst2c, zrows, ones)
  (P0,) = _sc_seg(xp, src2, dst2)
  h1 = _dense_layer(P0, C, xp, Wl0, bl0r, Wr0, True)
  (P1,) = _sc_seg(h1, src2, dst2)
  h2 = _dense_layer(P1, C, h1, Wl1, bl1r, Wr1, True)
  (P2,) = _sc_seg(h2, src2, dst2)
  return _final_layer(P2, C, h2, Wl2, bl2r, Wr2, Wo, bor)[:N]


# speedup vs baseline: 2.8407x; 1.0007x over previous
import functools

import jax
import jax.numpy as jnp
from jax import lax
from jax.experimental import pallas as pl
from jax.experimental.pallas import tpu as pltpu
from jax.experimental.pallas import tpu_sc as plsc

N = 10000
E = 320000
D = 128
OUT = 2

NC = 2
NS = 16
NW = NC * NS

CE = E // NW
B = 80
EP = 10240
K = EP // B
KP = K // 2
BC = 80
KC = CE // BC
NP = 10240
RPT = NP // NS
CPT = NP // NS


_MESH = plsc.VectorSubcoreMesh(core_axis_name="c", subcore_axis_name="s")


def _seg_body(h_hbm, src2, dst2, p_out, idx_s, idx_d, rows_a, rows_b,
              acc, sem_a, sem_b):
  cid = lax.axis_index("c")
  sid = lax.axis_index("s")
  wid = cid * NS + sid

  z16 = jnp.zeros((16,), jnp.float32)

  def zstore(i, _):
    rows_a[i // 8, pl.ds((i % 8) * 16, 16)] = z16
    return _

  lax.fori_loop(0, B * 8, zstore, None)
  for q in range(RPT // B):
    pltpu.sync_copy(rows_a, acc.at[pl.ds(sid * RPT + q * B, B), :])
  plsc.subcore_barrier()

  def gather(c, buf, sem):
    return pltpu.async_copy(h_hbm.at[idx_s.at[c]], buf, sem)

  def gwait(c, buf, sem):
    pltpu.make_async_copy(h_hbm.at[idx_s.at[c]], buf, sem).wait()

  def scatter(c, buf):
    pltpu.sync_copy(buf, acc.at[idx_d.at[c]], add=True)

  def pair(t, _):
    i = 2 * t
    gather(i + 1, rows_b, sem_b)
    gwait(i, rows_a, sem_a)
    scatter(i, rows_a)
    gather(i + 2, rows_a, sem_a)
    gwait(i + 1, rows_b, sem_b)
    scatter(i + 1, rows_b)
    return _

  for p in range(K // KP):
    pltpu.sync_copy(src2.at[wid, pl.ds(p * KP, KP), :], idx_s)
    pltpu.sync_copy(dst2.at[wid, pl.ds(p * KP, KP), :], idx_d)
    gather(0, rows_a, sem_a)
    lax.fori_loop(0, KP // 2 - 1, pair, None)
    gather(KP - 1, rows_b, sem_b)
    gwait(KP - 2, rows_a, sem_a)
    scatter(KP - 2, rows_a)
    gwait(KP - 1, rows_b, sem_b)
    scatter(KP - 1, rows_b)

  plsc.subcore_barrier()
  pltpu.sync_copy(acc.at[pl.ds(sid * RPT, RPT), :],
                  p_out.at[cid, pl.ds(sid * RPT, RPT), :])


_sc_seg = pl.kernel(
    _seg_body,
    out_type=(jax.ShapeDtypeStruct((NC, NP, D), jnp.float32),),
    mesh=_MESH,
    scratch_types=[
        pltpu.VMEM((KP, B), jnp.int32),
        pltpu.VMEM((KP, B), jnp.int32),
        pltpu.VMEM((B, D), jnp.float32),
        pltpu.VMEM((B, D), jnp.float32),
        pltpu.VMEM_SHARED((NP, D), jnp.float32),
        pltpu.SemaphoreType.DMA,
        pltpu.SemaphoreType.DMA,
    ])


WNC = 8


def _cnt_body(dst2, zrows, ones_hbm, c_out, idx_d, ones_v, cacc, sem):
  cid = lax.axis_index("c")
  sid = lax.axis_index("s")
  wid = cid * NS + sid

  pltpu.sync_copy(dst2.at[wid], idx_d)
  pltpu.sync_copy(zrows, cacc.at[pl.ds(sid * RPT, RPT), :])
  pltpu.sync_copy(ones_hbm, ones_v)
  plsc.subcore_barrier()

  def fire(c, _):
    pltpu.async_copy(ones_v, cacc.at[idx_d.at[c]], sem, add=True)
    return _

  def fire_drain(c, _):
    pltpu.async_copy(ones_v, cacc.at[idx_d.at[c + WNC]], sem, add=True)
    pltpu.make_async_copy(ones_v, cacc.at[idx_d.at[c]], sem).wait()
    return _

  def drain(c, _):
    pltpu.make_async_copy(ones_v, cacc.at[idx_d.at[c]], sem).wait()
    return _

  lax.fori_loop(0, WNC, fire, None)
  lax.fori_loop(0, KC - WNC, fire_drain, None)
  lax.fori_loop(KC - WNC, KC, drain, None)
  plsc.subcore_barrier()

  pltpu.sync_copy(cacc.at[pl.ds(sid * RPT, RPT), :],
                  c_out.at[cid, pl.ds(sid * RPT, RPT), :])


_sc_counts = pl.kernel(
    _cnt_body,
    out_type=(jax.ShapeDtypeStruct((NC, NP, D), jnp.float32),),
    mesh=_MESH,
    scratch_types=[
        pltpu.VMEM((KC, BC), jnp.int32),
        pltpu.VMEM((BC, D), jnp.float32),
        pltpu.VMEM_SHARED((NP, D), jnp.float32),
        pltpu.SemaphoreType.DMA,
    ])

R = 1024


def _dense0_body(p_ref, c_ref, h_ref, wl_ref, bl_ref, wr_ref, o_ref, inv_ref):
  cnt = c_ref[0][:, 0:1] + c_ref[1][:, 0:1]
  inv = 1.0 / jnp.maximum(cnt, 1.0)
  inv_ref[...] = inv
  agg = (p_ref[0] + p_ref[1]) * inv
  dn = (((1,), (1,)), ((), ()))
  acc = lax.dot_general(agg, wl_ref[...], dn,
                        precision=lax.Precision.HIGHEST,
                        preferred_element_type=jnp.float32)
  acc += lax.dot_general(h_ref[...], wr_ref[...], dn,
                         precision=lax.Precision.HIGHEST,
                         preferred_element_type=jnp.float32)
  acc += bl_ref[...]
  o_ref[...] = jnp.maximum(acc, 0.0)


def _dense_body(p_ref, inv_ref, h_ref, wl_ref, bl_ref, wr_ref, o_ref):
  agg = (p_ref[0] + p_ref[1]) * inv_ref[...]
  dn = (((1,), (1,)), ((), ()))
  acc = lax.dot_general(agg, wl_ref[...], dn,
                        precision=lax.Precision.HIGHEST,
                        preferred_element_type=jnp.float32)
  acc += lax.dot_general(h_ref[...], wr_ref[...], dn,
                         precision=lax.Precision.HIGHEST,
                         preferred_element_type=jnp.float32)
  acc += bl_ref[...]
  o_ref[...] = jnp.maximum(acc, 0.0)


def _final_body(p_ref, inv_ref, h_ref, wl_ref, bl_ref, wr_ref, wo_ref, bo_ref,
                o_ref):
  agg = (p_ref[0] + p_ref[1]) * inv_ref[...]
  dn = (((1,), (1,)), ((), ()))
  acc = lax.dot_general(agg, wl_ref[...], dn,
                        precision=lax.Precision.HIGHEST,
                        preferred_element_type=jnp.float32)
  acc += lax.dot_general(h_ref[...], wr_ref[...], dn,
                         precision=lax.Precision.HIGHEST,
                         preferred_element_type=jnp.float32)
  acc += bl_ref[...]
  o_ref[...] = lax.dot_general(acc, wo_ref[...], dn,
                               precision=lax.Precision.HIGHEST,
                               preferred_element_type=jnp.float32) + bo_ref[...]


_W_SPEC = pl.BlockSpec((D, D), lambda i: (0, 0))
_B_SPEC = pl.BlockSpec((1, D), lambda i: (0, 0))
_P_SPEC = pl.BlockSpec((NC, R, D), lambda i: (0, i, 0))
_C_SPEC = pl.BlockSpec((NC, R, D), lambda i: (0, i, 0))
_H_SPEC = pl.BlockSpec((R, D), lambda i: (i, 0))
_INV_SPEC = pl.BlockSpec((R, 1), lambda i: (i, 0))


def _dense0_layer(P, C, h, Wl, bl, Wr):
  return pl.pallas_call(
      _dense0_body,
      grid=(NP // R,),
      in_specs=[_P_SPEC, _C_SPEC, _H_SPEC, _W_SPEC, _B_SPEC, _W_SPEC],
      out_specs=[_H_SPEC, _INV_SPEC],
      out_shape=[jax.ShapeDtypeStruct((NP, D), jnp.float32),
                 jax.ShapeDtypeStruct((NP, 1), jnp.float32)],
  )(P, C, h, Wl, bl, Wr)


def _dense_layer(P, inv, h, Wl, bl, Wr):
  return pl.pallas_call(
      _dense_body,
      grid=(NP // R,),
      in_specs=[_P_SPEC, _INV_SPEC, _H_SPEC, _W_SPEC, _B_SPEC, _W_SPEC],
      out_specs=_H_SPEC,
      out_shape=jax.ShapeDtypeStruct((NP, D), jnp.float32),
  )(P, inv, h, Wl, bl, Wr)


def _final_layer(P, inv, h, Wl, bl, Wr, Wo, bo):
  return pl.pallas_call(
      _final_body,
      grid=(NP // R,),
      in_specs=[_P_SPEC, _INV_SPEC, _H_SPEC, _W_SPEC, _B_SPEC, _W_SPEC,
                pl.BlockSpec((OUT, D), lambda i: (0, 0)),
                pl.BlockSpec((1, OUT), lambda i: (0, 0))],
      out_specs=pl.BlockSpec((R, OUT), lambda i: (i, 0)),
      out_shape=jax.ShapeDtypeStruct((NP, OUT), jnp.float32),
  )(P, inv, h, Wl, bl, Wr, Wo, bo)


@jax.jit
def kernel(x, edge_index, Wl0, bl0, Wr0, Wl1, bl1, Wr1, Wl2, bl2, Wr2, Wo, bo):
  srcw = edge_index[0].reshape(NW, CE)
  dstw = edge_index[1].reshape(NW, CE)
  pad_iota = jnp.arange(EP - CE, dtype=jnp.int32)
  pad_src = jnp.broadcast_to(pad_iota[None, :] * 41 % N, (NW, EP - CE))
  pad_dst = jnp.broadcast_to(N + pad_iota[None, :], (NW, EP - CE))
  src2 = jnp.concatenate([srcw, pad_src], axis=1).reshape(NW, K, B)
  dst2 = jnp.concatenate([dstw, pad_dst], axis=1).reshape(NW, K, B)
  dst2c = edge_index[1].reshape(NW, KC, BC)
  zrows = jnp.zeros((RPT, D), jnp.float32)
  ones = jnp.ones((BC, D), jnp.float32)
  bl0r, bl1r, bl2r = bl0.reshape(1, D), bl1.reshape(1, D), bl2.reshape(1, D)
  bor = bo.reshape(1, OUT)
  xp = jnp.concatenate([x, jnp.zeros((NP - N, D), jnp.float32)], axis=0)

  (C,) = _sc_counts(dst2c, zrows, ones)
  (P0,) = _sc_seg(xp, src2, dst2)
  h1, inv = _dense0_layer(P0, C, xp, Wl0, bl0r, Wr0)
  (P1,) = _sc_seg(h1, src2, dst2)
  h2 = _dense_layer(P1, inv, h1, Wl1, bl1r, Wr1)
  (P2,) = _sc_seg(h2, src2, dst2)
  return _final_layer(P2, inv, h2, Wl2, bl2r, Wr2, Wo, bor)[:N]


# speedup vs baseline: 2.8427x; 1.0007x over previous
import jax
import jax.numpy as jnp
from jax import lax
from jax.experimental import pallas as pl
from jax.experimental.pallas import tpu as pltpu
from jax.experimental.pallas import tpu_sc as plsc

N = 10000
E = 320000
D = 128
OUT = 2

NC = 2
NS = 16
NW = NC * NS

CE = E // NW
B = 80
EP = 10240
K = EP // B
KP = K // 2
BC = 80
KC = CE // BC
NP = 10240
RPT = NP // NS
CPT = NP // NS


_MESH = plsc.VectorSubcoreMesh(core_axis_name="c", subcore_axis_name="s")


def _seg_body(h_hbm, src2, dst2, p_out, idx_s, idx_d, rows_a, rows_b,
              acc, sem_a, sem_b):
  cid = lax.axis_index("c")
  sid = lax.axis_index("s")
  wid = cid * NS + sid

  z16 = jnp.zeros((16,), jnp.float32)

  def zstore(i, _):
    rows_a[i // 8, pl.ds((i % 8) * 16, 16)] = z16
    return _

  lax.fori_loop(0, B * 8, zstore, None)
  for q in range(RPT // B):
    pltpu.sync_copy(rows_a, acc.at[pl.ds(sid * RPT + q * B, B), :])
  plsc.subcore_barrier()

  def gather(c, buf, sem):
    return pltpu.async_copy(h_hbm.at[idx_s.at[c]], buf, sem)

  def gwait(c, buf, sem):
    pltpu.make_async_copy(h_hbm.at[idx_s.at[c]], buf, sem).wait()

  def scatter(c, buf):
    pltpu.sync_copy(buf, acc.at[idx_d.at[c]], add=True)

  def pair(t, _):
    i = 2 * t
    gather(i + 1, rows_b, sem_b)
    gwait(i, rows_a, sem_a)
    scatter(i, rows_a)
    gather(i + 2, rows_a, sem_a)
    gwait(i + 1, rows_b, sem_b)
    scatter(i + 1, rows_b)
    return _

  for p in range(K // KP):
    pltpu.sync_copy(src2.at[wid, pl.ds(p * KP, KP), :], idx_s)
    pltpu.sync_copy(dst2.at[wid, pl.ds(p * KP, KP), :], idx_d)
    gather(0, rows_a, sem_a)
    lax.fori_loop(0, KP // 2 - 1, pair, None)
    gather(KP - 1, rows_b, sem_b)
    gwait(KP - 2, rows_a, sem_a)
    scatter(KP - 2, rows_a)
    gwait(KP - 1, rows_b, sem_b)
    scatter(KP - 1, rows_b)

  plsc.subcore_barrier()
  pltpu.sync_copy(acc.at[pl.ds(sid * RPT, RPT), :],
                  p_out.at[cid, pl.ds(sid * RPT, RPT), :])


_sc_seg = pl.kernel(
    _seg_body,
    out_type=(jax.ShapeDtypeStruct((NC, NP, D), jnp.float32),),
    mesh=_MESH,
    scratch_types=[
        pltpu.VMEM((KP, B), jnp.int32),
        pltpu.VMEM((KP, B), jnp.int32),
        pltpu.VMEM((B, D), jnp.float32),
        pltpu.VMEM((B, D), jnp.float32),
        pltpu.VMEM_SHARED((NP, D), jnp.float32),
        pltpu.SemaphoreType.DMA,
        pltpu.SemaphoreType.DMA,
    ])


WNC = 8


def _cnt_body(dst2, zrows, ones_hbm, c_out, idx_d, ones_v, cacc, sem):
  cid = lax.axis_index("c")
  sid = lax.axis_index("s")
  wid = cid * NS + sid

  pltpu.sync_copy(dst2.at[wid], idx_d)
  pltpu.sync_copy(zrows, cacc.at[pl.ds(sid * RPT, RPT), :])
  pltpu.sync_copy(ones_hbm, ones_v)
  plsc.subcore_barrier()

  def fire(c, _):
    pltpu.async_copy(ones_v, cacc.at[idx_d.at[c]], sem, add=True)
    return _

  def fire_drain(c, _):
    pltpu.async_copy(ones_v, cacc.at[idx_d.at[c + WNC]], sem, add=True)
    pltpu.make_async_copy(ones_v, cacc.at[idx_d.at[c]], sem).wait()
    return _

  def drain(c, _):
    pltpu.make_async_copy(ones_v, cacc.at[idx_d.at[c]], sem).wait()
    return _

  lax.fori_loop(0, WNC, fire, None)
  lax.fori_loop(0, KC - WNC, fire_drain, None)
  lax.fori_loop(KC - WNC, KC, drain, None)
  plsc.subcore_barrier()

  pltpu.sync_copy(cacc.at[pl.ds(sid * RPT, RPT), :],
                  c_out.at[cid, pl.ds(sid * RPT, RPT), :])


_sc_counts = pl.kernel(
    _cnt_body,
    out_type=(jax.ShapeDtypeStruct((NC, NP, D), jnp.float32),),
    mesh=_MESH,
    scratch_types=[
        pltpu.VMEM((KC, BC), jnp.int32),
        pltpu.VMEM((BC, D), jnp.float32),
        pltpu.VMEM_SHARED((NP, D), jnp.float32),
        pltpu.SemaphoreType.DMA,
    ])

R = 1024


def _dense0_body(p_ref, c_ref, h_ref, wl_ref, bl_ref, wr_ref, o_ref, inv_ref):
  cnt = c_ref[0][:, 0:1] + c_ref[1][:, 0:1]
  inv = 1.0 / jnp.maximum(cnt, 1.0)
  inv_ref[...] = inv
  agg = (p_ref[0] + p_ref[1]) * inv
  dn = (((1,), (1,)), ((), ()))
  acc = lax.dot_general(agg, wl_ref[...], dn,
                        precision=lax.Precision.HIGHEST,
                        preferred_element_type=jnp.float32)
  acc += lax.dot_general(h_ref[...], wr_ref[...], dn,
                         precision=lax.Precision.HIGHEST,
                         preferred_element_type=jnp.float32)
  acc += bl_ref[...]
  o_ref[...] = jnp.maximum(acc, 0.0)


def _dense_body(p_ref, inv_ref, h_ref, wl_ref, bl_ref, wr_ref, o_ref):
  agg = (p_ref[0] + p_ref[1]) * inv_ref[...]
  dn = (((1,), (1,)), ((), ()))
  acc = lax.dot_general(agg, wl_ref[...], dn,
                        precision=lax.Precision.HIGHEST,
                        preferred_element_type=jnp.float32)
  acc += lax.dot_general(h_ref[...], wr_ref[...], dn,
                         precision=lax.Precision.HIGHEST,
                         preferred_element_type=jnp.float32)
  acc += bl_ref[...]
  o_ref[...] = jnp.maximum(acc, 0.0)


def _final_body(p_ref, inv_ref, h_ref, wl_ref, bl_ref, wr_ref, wo_ref, bo_ref,
                o_ref):
  agg = (p_ref[0] + p_ref[1]) * inv_ref[...]
  dn = (((1,), (1,)), ((), ()))
  acc = lax.dot_general(agg, wl_ref[...], dn,
                        precision=lax.Precision.HIGHEST,
                        preferred_element_type=jnp.float32)
  acc += lax.dot_general(h_ref[...], wr_ref[...], dn,
                         precision=lax.Precision.HIGHEST,
                         preferred_element_type=jnp.float32)
  acc += bl_ref[...]
  o_ref[...] = lax.dot_general(acc, wo_ref[...], dn,
                               precision=lax.Precision.HIGHEST,
                               preferred_element_type=jnp.float32) + bo_ref[...]


_W_SPEC = pl.BlockSpec((D, D), lambda i: (0, 0))
_B_SPEC = pl.BlockSpec((1, D), lambda i: (0, 0))
_P_SPEC = pl.BlockSpec((NC, R, D), lambda i: (0, i, 0))
_C_SPEC = pl.BlockSpec((NC, R, D), lambda i: (0, i, 0))
_H_SPEC = pl.BlockSpec((R, D), lambda i: (i, 0))
_INV_SPEC = pl.BlockSpec((R, 1), lambda i: (i, 0))


def _dense0_layer(P, C, h, Wl, bl, Wr):
  return pl.pallas_call(
      _dense0_body,
      grid=(NP // R,),
      in_specs=[_P_SPEC, _C_SPEC, _H_SPEC, _W_SPEC, _B_SPEC, _W_SPEC],
      out_specs=[_H_SPEC, _INV_SPEC],
      out_shape=[jax.ShapeDtypeStruct((NP, D), jnp.float32),
                 jax.ShapeDtypeStruct((NP, 1), jnp.float32)],
  )(P, C, h, Wl, bl, Wr)


def _dense_layer(P, inv, h, Wl, bl, Wr):
  return pl.pallas_call(
      _dense_body,
      grid=(NP // R,),
      in_specs=[_P_SPEC, _INV_SPEC, _H_SPEC, _W_SPEC, _B_SPEC, _W_SPEC],
      out_specs=_H_SPEC,
      out_shape=jax.ShapeDtypeStruct((NP, D), jnp.float32),
  )(P, inv, h, Wl, bl, Wr)


def _final_layer(P, inv, h, Wl, bl, Wr, Wo, bo):
  return pl.pallas_call(
      _final_body,
      grid=(NP // R,),
      in_specs=[_P_SPEC, _INV_SPEC, _H_SPEC, _W_SPEC, _B_SPEC, _W_SPEC,
                pl.BlockSpec((OUT, D), lambda i: (0, 0)),
                pl.BlockSpec((1, OUT), lambda i: (0, 0))],
      out_specs=pl.BlockSpec((R, OUT), lambda i: (i, 0)),
      out_shape=jax.ShapeDtypeStruct((NP, OUT), jnp.float32),
  )(P, inv, h, Wl, bl, Wr, Wo, bo)


@jax.jit
def kernel(x, edge_index, Wl0, bl0, Wr0, Wl1, bl1, Wr1, Wl2, bl2, Wr2, Wo, bo):
  srcw = edge_index[0].reshape(NW, CE)
  dstw = edge_index[1].reshape(NW, CE)
  pad_iota = jnp.arange(EP - CE, dtype=jnp.int32)
  pad_src = jnp.broadcast_to(pad_iota[None, :] * 41 % N, (NW, EP - CE))
  pad_dst = jnp.broadcast_to(N + pad_iota[None, :], (NW, EP - CE))
  src2 = jnp.concatenate([srcw, pad_src], axis=1).reshape(NW, K, B)
  dst2 = jnp.concatenate([dstw, pad_dst], axis=1).reshape(NW, K, B)
  dst2c = edge_index[1].reshape(NW, KC, BC)
  zrows = jnp.zeros((RPT, D), jnp.float32)
  ones = jnp.ones((BC, D), jnp.float32)
  bl0r, bl1r, bl2r = bl0.reshape(1, D), bl1.reshape(1, D), bl2.reshape(1, D)
  bor = bo.reshape(1, OUT)
  xp = jnp.concatenate([x, jnp.zeros((NP - N, D), jnp.float32)], axis=0)

  (C,) = _sc_counts(dst2c, zrows, ones)
  (P0,) = _sc_seg(xp, src2, dst2)
  h1, inv = _dense0_layer(P0, C, xp, Wl0, bl0r, Wr0)
  (P1,) = _sc_seg(h1, src2, dst2)
  h2 = _dense_layer(P1, inv, h1, Wl1, bl1r, Wr1)
  (P2,) = _sc_seg(h2, src2, dst2)
  return _final_layer(P2, inv, h2, Wl2, bl2r, Wr2, Wo, bor)[:N]
